# baseline (speedup 1.0000x reference)
import functools

import jax
import jax.numpy as jnp
from jax.experimental import pallas as pl
from jax.experimental.pallas import tpu as pltpu

_WEIGHT = 0.01


def _chamfer_body(x_ref, y_ref, o_ref, yb_ref, y2g_ref, colmin_ref, *,
                  n_blocks, tile_n, n, m):
    nb = pl.program_id(1)

    ones8 = jnp.ones((128, 8), jnp.float32)

    @pl.when(nb == 0)
    def _():
        yf = y_ref[0]
        yb_ref[:, :] = yf.astype(jnp.bfloat16)
        y2 = jax.lax.dot_general(
            yf * yf, ones8, (((1,), (0,)), ((), ())),
            preferred_element_type=jnp.float32)[:, 0:1]
        y2g_ref[:, :] = y2.reshape(m // 128, 128)

    xf = x_ref[0]
    xs = (-2.0 * xf).astype(jnp.bfloat16)
    x2 = jax.lax.dot_general(
        xf * xf, ones8, (((1,), (0,)), ((), ())),
        preferred_element_type=jnp.float32)[:, 0:1]

    chunk = 512
    gm = None
    for c in range(m // chunk):
        t = jax.lax.dot_general(
            xs, yb_ref[c * chunk:(c + 1) * chunk, :], (((1,), (1,)), ((), ())),
            preferred_element_type=jnp.float32)

        bc = jnp.min(t + x2, axis=0)

        @pl.when(nb == 0)
        def _():
            colmin_ref[0, c * chunk:(c + 1) * chunk] = bc

        @pl.when(nb > 0)
        def _():
            colmin_ref[0, c * chunk:(c + 1) * chunk] = jnp.minimum(
                colmin_ref[0, c * chunk:(c + 1) * chunk], bc)

        for g in range(chunk // 128):
            gi = c * (chunk // 128) + g
            part = t[:, g * 128:(g + 1) * 128] + y2g_ref[gi:gi + 1, :]
            gm = part if gm is None else jnp.minimum(gm, part)

    @pl.when(nb == 0)
    def _():
        o_ref[0, 0, :] = jnp.zeros((128,), jnp.float32)

    rowmin = jnp.min(gm, axis=1)
    cham_x = jnp.maximum(rowmin + x2[:, 0], 0.0)
    o_ref[0, 0, :] += jnp.full((128,), jnp.sum(cham_x) * (1.0 / n))

    @pl.when(nb == n_blocks - 1)
    def _():
        cham_y = jnp.maximum(
            colmin_ref[0, :] + y2g_ref[:, :].reshape(-1), 0.0)
        o_ref[0, 0, :] += jnp.full((128,), jnp.sum(cham_y) * (1.0 / m))


def kernel(transformed_source, transformed_target):
    x = transformed_source.astype(jnp.float32)
    y = transformed_target.astype(jnp.float32)
    b, n, d = x.shape
    _, m, _ = y.shape

    tile_n = 512
    n_blocks = n // tile_n

    body = functools.partial(
        _chamfer_body, n_blocks=n_blocks, tile_n=tile_n, n=n, m=m)

    out = pl.pallas_call(
        body,
        grid=(b, n_blocks),
        in_specs=[
            pl.BlockSpec((1, tile_n, d), lambda bi, ni: (bi, ni, 0)),
            pl.BlockSpec((1, m, d), lambda bi, ni: (bi, 0, 0)),
        ],
        out_specs=pl.BlockSpec((1, 1, 128), lambda bi, ni: (bi, 0, 0)),
        out_shape=jax.ShapeDtypeStruct((b, 1, 128), jnp.float32),
        scratch_shapes=[
            pltpu.VMEM((m, d), jnp.bfloat16),
            pltpu.VMEM((m // 128, 128), jnp.float32),
            pltpu.VMEM((1, m), jnp.float32),
        ],
    )(x, y)

    return _WEIGHT * jnp.mean(out[:, 0, 0])


# speedup vs baseline: 1.4319x; 1.4319x over previous
import functools

import jax
import jax.numpy as jnp
from jax.experimental import pallas as pl
from jax.experimental.pallas import tpu as pltpu

_WEIGHT = 0.01


def _chamfer_body(x_ref, y_ref, o_ref, yb_ref, y2g_ref, colmin_ref, *,
                  n_blocks, tile_n, n, m):
    nb = pl.program_id(1)

    ones8 = jnp.ones((128, 8), jnp.float32)

    @pl.when(nb == 0)
    def _():
        yf = y_ref[0]
        yb_ref[:, :] = yf.astype(jnp.bfloat16)
        y2 = jax.lax.dot_general(
            yf * yf, ones8, (((1,), (0,)), ((), ())),
            preferred_element_type=jnp.float32)[:, 0:1]
        y2g_ref[:, :] = y2.reshape(m // 128, 128)

    xf = x_ref[0]
    xs = (-2.0 * xf).astype(jnp.bfloat16)
    x2 = jax.lax.dot_general(
        xf * xf, ones8, (((1,), (0,)), ((), ())),
        preferred_element_type=jnp.float32)[:, 0:1]

    chunk = 512
    gm = None
    for c in range(m // chunk):
        t = jax.lax.dot_general(
            xs, yb_ref[c * chunk:(c + 1) * chunk, :], (((1,), (1,)), ((), ())),
            preferred_element_type=jnp.float32)

        bc = jnp.min(t + x2, axis=0)

        @pl.when(nb == 0)
        def _():
            colmin_ref[0, c * chunk:(c + 1) * chunk] = bc

        @pl.when(nb > 0)
        def _():
            colmin_ref[0, c * chunk:(c + 1) * chunk] = jnp.minimum(
                colmin_ref[0, c * chunk:(c + 1) * chunk], bc)

        for g in range(chunk // 128):
            gi = c * (chunk // 128) + g
            part = t[:, g * 128:(g + 1) * 128] + y2g_ref[gi:gi + 1, :]
            gm = part if gm is None else jnp.minimum(gm, part)

    @pl.when(nb == 0)
    def _():
        o_ref[0, 0, :] = jnp.zeros((128,), jnp.float32)

    rowmin = jnp.min(gm, axis=1)
    cham_x = jnp.maximum(rowmin + x2[:, 0], 0.0)
    o_ref[0, 0, :] += jnp.full((128,), jnp.sum(cham_x) * (1.0 / n))

    @pl.when(nb == n_blocks - 1)
    def _():
        cham_y = jnp.maximum(
            colmin_ref[0, :] + y2g_ref[:, :].reshape(-1), 0.0)
        o_ref[0, 0, :] += jnp.full((128,), jnp.sum(cham_y) * (1.0 / m))


def kernel(transformed_source, transformed_target):
    x = transformed_source.astype(jnp.float32)
    y = transformed_target.astype(jnp.float32)
    b, n, d = x.shape
    _, m, _ = y.shape

    tile_n = 2048
    n_blocks = n // tile_n

    body = functools.partial(
        _chamfer_body, n_blocks=n_blocks, tile_n=tile_n, n=n, m=m)

    out = pl.pallas_call(
        body,
        grid=(b, n_blocks),
        in_specs=[
            pl.BlockSpec((1, tile_n, d), lambda bi, ni: (bi, ni, 0)),
            pl.BlockSpec((1, m, d), lambda bi, ni: (bi, 0, 0)),
        ],
        out_specs=pl.BlockSpec((1, 1, 128), lambda bi, ni: (bi, 0, 0)),
        out_shape=jax.ShapeDtypeStruct((b, 1, 128), jnp.float32),
        scratch_shapes=[
            pltpu.VMEM((m, d), jnp.bfloat16),
            pltpu.VMEM((m // 128, 128), jnp.float32),
            pltpu.VMEM((1, m), jnp.float32),
        ],
    )(x, y)

    return _WEIGHT * jnp.mean(out[:, 0, 0])
